# baseline (speedup 1.0000x reference)
import functools

import jax
import jax.numpy as jnp
from jax import lax
from jax.experimental import pallas as pl
from jax.experimental.pallas import tpu as pltpu
from jax.experimental.pallas import tpu_sc as plsc

_VOCAB = 100000
_D = 128
_B = 4096
_H = 50
_N = _B * _H
_ROWS = 64
_PER_PLANE = 128 // _ROWS
_RING = 10
_LOOK = 4


@functools.lru_cache(maxsize=None)
def _build(nc, ns):
    nw = nc * ns
    bat_w = _B // nw
    per_w = _H * bat_w
    nchunk = _H * _PER_PLANE
    assert nchunk % _RING == 0
    mesh = plsc.VectorSubcoreMesh(core_axis_name="c", subcore_axis_name="s")

    @functools.partial(
        pl.kernel,
        mesh=mesh,
        out_type=jax.ShapeDtypeStruct((_H, _B, _D), jnp.float32),
        scratch_types=(
            [pltpu.VMEM((per_w,), jnp.int32)]
            + [pltpu.VMEM((_ROWS, _D), jnp.float32) for _ in range(_RING)]
            + [pltpu.SemaphoreType.DMA for _ in range(2 * _RING)]
        ),
    )
    def k(x_hbm, table_hbm, out_hbm, idx_v, *scratch):
        bufs = scratch[:_RING]
        semg = scratch[_RING:2 * _RING]
        sems = scratch[2 * _RING:]
        wid = lax.axis_index("s") * nc + lax.axis_index("c")
        pltpu.sync_copy(x_hbm.at[wid], idx_v)
        b_base = wid * bat_w

        def gather(c, slot):
            return pltpu.make_async_copy(
                table_hbm.at[idx_v.at[pl.ds(c * _ROWS, _ROWS)]],
                bufs[slot], semg[slot])

        def scatter(c, slot):
            return pltpu.make_async_copy(
                bufs[slot],
                out_hbm.at[c // _PER_PLANE,
                           pl.ds(b_base + (c % _PER_PLANE) * _ROWS, _ROWS)],
                sems[slot])

        for c in range(_LOOK):
            gather(c, c).start()

        def outer(o, carry):
            for b in range(_RING):
                c = o * _RING + b
                gb = (b + _LOOK) % _RING
                @pl.when(c >= _RING - _LOOK)
                def _():
                    scatter(c - (_RING - _LOOK), gb).wait()

                @pl.when(c + _LOOK < nchunk)
                def _():
                    gather(c + _LOOK, gb).start()

                gather(c, b).wait()
                scatter(c, b).start()
            return carry

        lax.fori_loop(0, nchunk // _RING, outer, 0)
        for c in range(nchunk - (_RING - _LOOK), nchunk):
            scatter(c, c % _RING).wait()

    return k


def kernel(x, table):
    info = plsc.get_sparse_core_info()
    nc, ns = info.num_cores, info.num_subcores
    nw = nc * ns
    bat_w = _B // nw
    xi = (x.astype(jnp.int32).T.reshape(_H, nw, bat_w)
          .transpose(1, 0, 2).reshape(nw, _H * bat_w))
    out = _build(nc, ns)(xi, table)
    return out.transpose(1, 0, 2)


# speedup vs baseline: 1.0010x; 1.0010x over previous
import functools

import jax
import jax.numpy as jnp
from jax import lax
from jax.experimental import pallas as pl
from jax.experimental.pallas import tpu as pltpu
from jax.experimental.pallas import tpu_sc as plsc

_VOCAB = 100000
_D = 128
_B = 4096
_H = 50
_N = _B * _H
_RING = 5
_LOOK = 2


@functools.lru_cache(maxsize=None)
def _build(nc, ns):
    nw = nc * ns
    bat_w = _B // nw
    per_w = _H * bat_w
    nchunk = _H
    assert nchunk % _RING == 0
    mesh = plsc.VectorSubcoreMesh(core_axis_name="c", subcore_axis_name="s")

    @functools.partial(
        pl.kernel,
        mesh=mesh,
        out_type=jax.ShapeDtypeStruct((_H, _B, _D), jnp.float32),
        scratch_types=(
            [pltpu.VMEM((per_w,), jnp.int32)]
            + [pltpu.VMEM((bat_w, _D), jnp.float32) for _ in range(_RING)]
            + [pltpu.SemaphoreType.DMA for _ in range(2 * _RING)]
        ),
    )
    def k(x_hbm, table_hbm, out_hbm, idx_v, *scratch):
        bufs = scratch[:_RING]
        semg = scratch[_RING:2 * _RING]
        sems = scratch[2 * _RING:]
        wid = lax.axis_index("s") * nc + lax.axis_index("c")
        pltpu.sync_copy(x_hbm.at[wid], idx_v)
        b_base = wid * bat_w

        def gather(c, slot):
            return pltpu.make_async_copy(
                table_hbm.at[idx_v.at[pl.ds(c * bat_w, bat_w)]],
                bufs[slot], semg[slot])

        def scatter(c, slot):
            return pltpu.make_async_copy(
                bufs[slot],
                out_hbm.at[c, pl.ds(b_base, bat_w)],
                sems[slot])

        for c in range(_LOOK):
            gather(c, c).start()

        def outer(o, carry):
            for b in range(_RING):
                c = o * _RING + b
                gb = (b + _LOOK) % _RING
                @pl.when(c >= _RING - _LOOK)
                def _():
                    scatter(c - (_RING - _LOOK), gb).wait()

                @pl.when(c + _LOOK < nchunk)
                def _():
                    gather(c + _LOOK, gb).start()

                gather(c, b).wait()
                scatter(c, b).start()
            return carry

        lax.fori_loop(0, nchunk // _RING, outer, 0)
        for c in range(nchunk - (_RING - _LOOK), nchunk):
            scatter(c, c % _RING).wait()

    return k


def kernel(x, table):
    info = plsc.get_sparse_core_info()
    nc, ns = info.num_cores, info.num_subcores
    nw = nc * ns
    bat_w = _B // nw
    xi = (x.astype(jnp.int32).T.reshape(_H, nw, bat_w)
          .transpose(1, 0, 2).reshape(nw, _H * bat_w))
    out = _build(nc, ns)(xi, table)
    return out.transpose(1, 0, 2)
